# Initial kernel scaffold; baseline (speedup 1.0000x reference)
import jax
import jax.numpy as jnp
from jax.experimental import pallas as pl


def kernel(user_emb, item_emb, user_prototypes, item_prototypes, adj_indices, adj_values):
    raise NotImplementedError("write your pallas kernel here")


# speedup vs baseline: 2.7708x; 2.7708x over previous
import functools

import jax
import jax.numpy as jnp
from jax import lax
from jax.experimental import pallas as pl
from jax.experimental.pallas import tpu as pltpu
from jax.experimental.pallas import tpu_sc as plsc

USER_N = 25000
ITEM_N = 25000
NODES = USER_N + ITEM_N
EMB = 64
HALF = EMB // 2
LAYERS = 3
EDGES = 800000
NC = 2
NS = 16
CHUNK = 128
NCH = -(-EDGES // (NS * CHUNK))
EPT = NCH * CHUNK
E_PAD = EPT * NS
STRIPE = NODES // NS
WB = 125
NWB = STRIPE // WB


def _zero2d(ref, nrows):
    def body(r, _):
        ref[r, pl.ds(0, 16)] = jnp.zeros((16,), jnp.float32)
        ref[r, pl.ds(16, 16)] = jnp.zeros((16,), jnp.float32)
        return 0
    lax.fori_loop(0, nrows, body, 0)


def _sc_body(x0, cols2, rows, vals, outm, x1, x2, x3,
             acc, ga, colsv, rowsv, valsv, zer, stage, sem):
    c = lax.axis_index("c")
    s = lax.axis_index("s")
    row0 = s * STRIPE
    ebase = s * EPT

    _zero2d(zer, WB)

    xs_in = (x0, x1, x2)
    xs_out = (x1, x2, x3)
    for l in range(LAYERS):
        xi = xs_in[l]
        xo = xs_out[l]
        for k in range(NWB):
            pltpu.sync_copy(zer, acc.at[pl.ds(row0 + k * WB, WB)])
        plsc.subcore_barrier()

        def chunk(i, _):
            off = ebase + i * CHUNK
            pltpu.sync_copy(cols2.at[c, pl.ds(off, CHUNK)], colsv)
            pltpu.sync_copy(rows.at[pl.ds(off, CHUNK)], rowsv)
            pltpu.sync_copy(vals.at[pl.ds(off, CHUNK)], valsv)
            pltpu.async_copy(xi.at[colsv], ga, sem).wait()

            def mulb(j, _):
                vv = plsc.load_gather(valsv, [jnp.full((16,), j, jnp.int32)])
                ga[j, pl.ds(0, 16)] = ga[j, pl.ds(0, 16)] * vv
                ga[j, pl.ds(16, 16)] = ga[j, pl.ds(16, 16)] * vv
                return 0
            lax.fori_loop(0, CHUNK, mulb, 0)

            pltpu.sync_copy(ga, acc.at[rowsv], add=True)
            return 0
        lax.fori_loop(0, NCH, chunk, 0)
        plsc.subcore_barrier()

        for k in range(NWB):
            b = row0 + k * WB
            pltpu.sync_copy(acc.at[pl.ds(b, WB)], stage)
            pltpu.sync_copy(stage, xo.at[pl.ds(c * NODES + b, WB)])

    for k in range(NWB):
        b = c * NODES + row0 + k * WB
        pltpu.sync_copy(x0.at[pl.ds(b, WB)], stage)
        for xb in (x1, x2, x3):
            pltpu.sync_copy(xb.at[pl.ds(b, WB)], zer)

            def addb(r, _):
                stage[r, pl.ds(0, 16)] = stage[r, pl.ds(0, 16)] + zer[r, pl.ds(0, 16)]
                stage[r, pl.ds(16, 16)] = stage[r, pl.ds(16, 16)] + zer[r, pl.ds(16, 16)]
                return 0
            lax.fori_loop(0, WB, addb, 0)

        def scaleb(r, _):
            q = jnp.float32(0.25)
            stage[r, pl.ds(0, 16)] = stage[r, pl.ds(0, 16)] * q
            stage[r, pl.ds(16, 16)] = stage[r, pl.ds(16, 16)] * q
            return 0
        lax.fori_loop(0, WB, scaleb, 0)
        pltpu.sync_copy(stage, outm.at[pl.ds(b, WB)])


@jax.jit
def _lgcn_sc(x0, cols2, rows, vals):
    mesh = plsc.VectorSubcoreMesh(core_axis_name="c", subcore_axis_name="s")
    f = pl.kernel(
        _sc_body,
        out_type=[
            jax.ShapeDtypeStruct((2 * NODES, HALF), jnp.float32),
            jax.ShapeDtypeStruct((2 * NODES, HALF), jnp.float32),
            jax.ShapeDtypeStruct((2 * NODES, HALF), jnp.float32),
            jax.ShapeDtypeStruct((2 * NODES, HALF), jnp.float32),
        ],
        mesh=mesh,
        scratch_types=[
            pltpu.VMEM_SHARED((NODES, HALF), jnp.float32),
            pltpu.VMEM((CHUNK, HALF), jnp.float32),
            pltpu.VMEM((CHUNK,), jnp.int32),
            pltpu.VMEM((CHUNK,), jnp.int32),
            pltpu.VMEM((CHUNK,), jnp.float32),
            pltpu.VMEM((WB, HALF), jnp.float32),
            pltpu.VMEM((WB, HALF), jnp.float32),
            pltpu.SemaphoreType.DMA,
        ],
        compiler_params=pltpu.CompilerParams(
            use_tc_tiling_on_sc=False, needs_layout_passes=False),
    )
    return f(x0, cols2, rows, vals)


def kernel(user_emb, item_emb, user_prototypes, item_prototypes, adj_indices, adj_values):
    ego = jnp.concatenate([user_emb, item_emb], axis=0)
    x0 = jnp.concatenate([ego[:, :HALF], ego[:, HALF:]], axis=0)
    rows = adj_indices[0]
    cols = adj_indices[1]
    pad = E_PAD - EDGES
    rows_p = jnp.concatenate([rows, jnp.zeros((pad,), jnp.int32)])
    cols_p = jnp.concatenate([cols, jnp.zeros((pad,), jnp.int32)])
    vals_p = jnp.concatenate([adj_values, jnp.zeros((pad,), jnp.float32)])
    cols2 = jnp.stack([cols_p, cols_p + NODES])

    outm = _lgcn_sc(x0, cols2, rows_p, vals_p)[0]

    lgcn = jnp.concatenate([outm[:NODES], outm[NODES:]], axis=1)
    user_all = lgcn[:USER_N]
    item_all = lgcn[USER_N:]
    return (user_all, item_all, user_emb, item_emb,
            user_prototypes, item_prototypes)


# speedup vs baseline: 6.5210x; 2.3535x over previous
import jax
import jax.numpy as jnp
from jax import lax
from jax.experimental import pallas as pl
from jax.experimental.pallas import tpu as pltpu
from jax.experimental.pallas import tpu_sc as plsc

USER_N = 25000
ITEM_N = 25000
NODES = USER_N + ITEM_N
EMB = 64
HALF = EMB // 2
LAYERS = 3
EDGES = 800000
NC = 2
NS = 16
CHUNK = 128
BLK = 2 * CHUNK
NB = -(-EDGES // (NS * BLK))
EPT = NB * BLK
E_PAD = EPT * NS
NCHT = E_PAD // CHUNK
STRIPE = NODES // NS
WB = 125
NWB = STRIPE // WB


def _zero2d(ref, nrows):
    def body(r, _):
        ref[r, pl.ds(0, 16)] = jnp.zeros((16,), jnp.float32)
        ref[r, pl.ds(16, 16)] = jnp.zeros((16,), jnp.float32)
        return 0
    lax.fori_loop(0, nrows, body, 0)


def _sc_body(x0, cols3, rows2, vals, outm, x1, x2, x3,
             acc, ga0, ga1, cb0, cb1, rb0, rb1, vb0, vb1,
             zer, stage, sem0, sem1):
    c = lax.axis_index("c")
    s = lax.axis_index("s")
    row0 = s * STRIPE
    cbase = s * (EPT // CHUNK)

    _zero2d(zer, WB)

    gas = (ga0, ga1)
    cbs = (cb0, cb1)
    rbs = (rb0, rb1)
    vbs = (vb0, vb1)
    sems = (sem0, sem1)

    def load_fire(xi, i, p):
        blk = cbase + i * 2
        pltpu.sync_copy(cols3.at[c, pl.ds(blk, 2)], cbs[p])
        pltpu.sync_copy(rows2.at[pl.ds(blk, 2)], rbs[p])
        pltpu.sync_copy(vals.at[pl.ds(blk * CHUNK, BLK)], vbs[p])
        pltpu.async_copy(xi.at[cbs[p].at[0]], gas[p].at[pl.ds(0, CHUNK)], sems[p])
        pltpu.async_copy(xi.at[cbs[p].at[1]], gas[p].at[pl.ds(CHUNK, CHUNK)], sems[p])

    def drain(xi, p):
        pltpu.make_async_copy(xi.at[cbs[p].at[0]], gas[p].at[pl.ds(0, CHUNK)], sems[p]).wait()
        pltpu.make_async_copy(xi.at[cbs[p].at[1]], gas[p].at[pl.ds(CHUNK, CHUNK)], sems[p]).wait()

    def scale(p):
        ga = gas[p]
        vb = vbs[p]

        def grp(g, _):
            vv = vb[pl.ds(g * 16, 16)]
            for e in range(16):
                j = g * 16 + e
                bcast = vv[jnp.full((16,), e, jnp.int32)]
                ga[j, pl.ds(0, 16)] = ga[j, pl.ds(0, 16)] * bcast
                ga[j, pl.ds(16, 16)] = ga[j, pl.ds(16, 16)] * bcast
            return 0
        lax.fori_loop(0, BLK // 16, grp, 0)

    def scat(p):
        pltpu.sync_copy(gas[p].at[pl.ds(0, CHUNK)], acc.at[rbs[p].at[0]], add=True)
        pltpu.sync_copy(gas[p].at[pl.ds(CHUNK, CHUNK)], acc.at[rbs[p].at[1]], add=True)

    xs_in = (x0, x1, x2)
    xs_out = (x1, x2, x3)
    for l in range(LAYERS):
        xi = xs_in[l]
        xo = xs_out[l]
        for k in range(NWB):
            pltpu.sync_copy(zer, acc.at[pl.ds(row0 + k * WB, WB)])
        plsc.subcore_barrier()

        load_fire(xi, 0, 0)

        def body2(i2, _):
            i = 2 * i2
            load_fire(xi, i + 1, 1)
            drain(xi, 0)
            scale(0)
            scat(0)

            @pl.when(i2 + 1 < NB // 2)
            def _():
                load_fire(xi, i + 2, 0)
            drain(xi, 1)
            scale(1)
            scat(1)
            return 0
        lax.fori_loop(0, NB // 2, body2, 0)
        plsc.subcore_barrier()

        for k in range(NWB):
            b = row0 + k * WB
            pltpu.sync_copy(acc.at[pl.ds(b, WB)], stage)
            pltpu.sync_copy(stage, xo.at[pl.ds(c * NODES + b, WB)])

    for k in range(NWB):
        b = c * NODES + row0 + k * WB
        pltpu.sync_copy(x0.at[pl.ds(b, WB)], stage)
        for xb in (x1, x2, x3):
            pltpu.sync_copy(xb.at[pl.ds(b, WB)], zer)

            def addb(r, _):
                stage[r, pl.ds(0, 16)] = stage[r, pl.ds(0, 16)] + zer[r, pl.ds(0, 16)]
                stage[r, pl.ds(16, 16)] = stage[r, pl.ds(16, 16)] + zer[r, pl.ds(16, 16)]
                return 0
            lax.fori_loop(0, WB, addb, 0)

        def scaleb(r, _):
            q = jnp.float32(0.25)
            stage[r, pl.ds(0, 16)] = stage[r, pl.ds(0, 16)] * q
            stage[r, pl.ds(16, 16)] = stage[r, pl.ds(16, 16)] * q
            return 0
        lax.fori_loop(0, WB, scaleb, 0)
        pltpu.sync_copy(stage, outm.at[pl.ds(b, WB)])


@jax.jit
def _lgcn_sc(x0, cols3, rows2, vals):
    mesh = plsc.VectorSubcoreMesh(core_axis_name="c", subcore_axis_name="s")
    f = pl.kernel(
        _sc_body,
        out_type=[
            jax.ShapeDtypeStruct((2 * NODES, HALF), jnp.float32),
            jax.ShapeDtypeStruct((2 * NODES, HALF), jnp.float32),
            jax.ShapeDtypeStruct((2 * NODES, HALF), jnp.float32),
            jax.ShapeDtypeStruct((2 * NODES, HALF), jnp.float32),
        ],
        mesh=mesh,
        scratch_types=[
            pltpu.VMEM_SHARED((NODES, HALF), jnp.float32),
            pltpu.VMEM((BLK, HALF), jnp.float32),
            pltpu.VMEM((BLK, HALF), jnp.float32),
            pltpu.VMEM((2, CHUNK), jnp.int32),
            pltpu.VMEM((2, CHUNK), jnp.int32),
            pltpu.VMEM((2, CHUNK), jnp.int32),
            pltpu.VMEM((2, CHUNK), jnp.int32),
            pltpu.VMEM((BLK,), jnp.float32),
            pltpu.VMEM((BLK,), jnp.float32),
            pltpu.VMEM((WB, HALF), jnp.float32),
            pltpu.VMEM((WB, HALF), jnp.float32),
            pltpu.SemaphoreType.DMA,
            pltpu.SemaphoreType.DMA,
        ],
        compiler_params=pltpu.CompilerParams(
            use_tc_tiling_on_sc=False, needs_layout_passes=False),
    )
    return f(x0, cols3, rows2, vals)


def kernel(user_emb, item_emb, user_prototypes, item_prototypes, adj_indices, adj_values):
    ego = jnp.concatenate([user_emb, item_emb], axis=0)
    x0 = jnp.concatenate([ego[:, :HALF], ego[:, HALF:]], axis=0)
    rows = adj_indices[0]
    cols = adj_indices[1]
    pad = E_PAD - EDGES
    rows_p = jnp.concatenate([rows, jnp.zeros((pad,), jnp.int32)])
    cols_p = jnp.concatenate([cols, jnp.zeros((pad,), jnp.int32)])
    vals_p = jnp.concatenate([adj_values, jnp.zeros((pad,), jnp.float32)])
    cols3 = jnp.stack([cols_p, cols_p + NODES]).reshape(2, NCHT, CHUNK)
    rows2 = rows_p.reshape(NCHT, CHUNK)

    outm = _lgcn_sc(x0, cols3, rows2, vals_p)[0]

    lgcn = jnp.concatenate([outm[:NODES], outm[NODES:]], axis=1)
    user_all = lgcn[:USER_N]
    item_all = lgcn[USER_N:]
    return (user_all, item_all, user_emb, item_emb,
            user_prototypes, item_prototypes)


# speedup vs baseline: 6.8236x; 1.0464x over previous
import jax
import jax.numpy as jnp
from jax import lax
from jax.experimental import pallas as pl
from jax.experimental.pallas import tpu as pltpu
from jax.experimental.pallas import tpu_sc as plsc

USER_N = 25000
ITEM_N = 25000
NODES = USER_N + ITEM_N
EMB = 64
HALF = EMB // 2
LAYERS = 3
EDGES = 800000
NC = 2
NS = 16
CHUNK = 128
BLK = 2 * CHUNK
NCHT = EDGES // CHUNK
NB = 195
NTAIL = NCHT - NS * 2 * NB
STRIPE = NODES // NS
WB = 125
NWB = STRIPE // WB


def _zero2d(ref, nrows):
    def body(r, _):
        ref[r, pl.ds(0, 16)] = jnp.zeros((16,), jnp.float32)
        ref[r, pl.ds(16, 16)] = jnp.zeros((16,), jnp.float32)
        return 0
    lax.fori_loop(0, nrows, body, 0)


def _sc_body(x0, cols1, rows3, vals1, outm, x1, x2, x3,
             acc, ga0, ga1, cb0, cb1, rb0, rb1, vb0, vb1,
             zer, stage, gsem0, gsem1, ssem0, ssem1, msem):
    c = lax.axis_index("c")
    s = lax.axis_index("s")
    row0 = s * STRIPE
    cbase = s * (2 * NB) + jnp.minimum(s, NTAIL)
    coff = jnp.full((16,), c * NODES, jnp.int32)

    _zero2d(zer, WB)

    gas = (ga0, ga1)
    cbs = (cb0, cb1)
    rbs = (rb0, rb1)
    vbs = (vb0, vb1)
    gsems = (gsem0, gsem1)
    ssems = (ssem0, ssem1)

    def drain_scat(p):
        pltpu.make_async_copy(
            gas[p].at[pl.ds(0, CHUNK)], acc.at[rbs[p].at[0]], ssems[p]).wait()
        pltpu.make_async_copy(
            gas[p].at[pl.ds(CHUNK, CHUNK)], acc.at[rbs[p].at[1]], ssems[p]).wait()

    def fire(xi, j, p, first):
        if not first:
            drain_scat(p)
        blk = cbase + 2 * j
        off = blk * CHUNK
        pltpu.sync_copy(cols1.at[pl.ds(off, BLK)], cbs[p])
        for g in range(BLK // 16):
            cbs[p][pl.ds(g * 16, 16)] = cbs[p][pl.ds(g * 16, 16)] + coff
        pltpu.sync_copy(rows3.at[pl.ds(blk, 2)], rbs[p])
        pltpu.sync_copy(vals1.at[pl.ds(off, BLK)], vbs[p])
        pltpu.async_copy(
            xi.at[cbs[p].at[pl.ds(0, CHUNK)]], gas[p].at[pl.ds(0, CHUNK)], gsems[p])
        pltpu.async_copy(
            xi.at[cbs[p].at[pl.ds(CHUNK, CHUNK)]], gas[p].at[pl.ds(CHUNK, CHUNK)], gsems[p])

    def scale(ga, vb, nedge):
        def grp(g, _):
            vv = vb[pl.ds(g * 16, 16)]
            for e in range(16):
                j = g * 16 + e
                bcast = vv[jnp.full((16,), e, jnp.int32)]
                ga[j, pl.ds(0, 16)] = ga[j, pl.ds(0, 16)] * bcast
                ga[j, pl.ds(16, 16)] = ga[j, pl.ds(16, 16)] * bcast
            return 0
        lax.fori_loop(0, nedge // 16, grp, 0)

    def process(xi, p):
        pltpu.make_async_copy(
            xi.at[cbs[p].at[pl.ds(0, CHUNK)]], gas[p].at[pl.ds(0, CHUNK)], gsems[p]).wait()
        pltpu.make_async_copy(
            xi.at[cbs[p].at[pl.ds(CHUNK, CHUNK)]], gas[p].at[pl.ds(CHUNK, CHUNK)], gsems[p]).wait()
        scale(gas[p], vbs[p], BLK)
        pltpu.async_copy(
            gas[p].at[pl.ds(0, CHUNK)], acc.at[rbs[p].at[0]], ssems[p], add=True)
        pltpu.async_copy(
            gas[p].at[pl.ds(CHUNK, CHUNK)], acc.at[rbs[p].at[1]], ssems[p], add=True)

    xs_in = (x0, x1, x2)
    xs_out = (x1, x2, x3)
    for l in range(LAYERS):
        xi = xs_in[l]
        xo = xs_out[l]
        for k in range(NWB):
            pltpu.async_copy(zer, acc.at[pl.ds(row0 + k * WB, WB)], gsem0)
        for k in range(NWB):
            pltpu.make_async_copy(zer, acc.at[pl.ds(row0 + k * WB, WB)], gsem0).wait()
        plsc.subcore_barrier()

        fire(xi, 0, 0, True)
        fire(xi, 1, 1, True)
        process(xi, 0)

        def body(i2, _):
            j = 2 * i2
            fire(xi, j + 2, 0, False)
            process(xi, 1)
            fire(xi, j + 3, 1, False)
            process(xi, 0)
            return 0
        lax.fori_loop(0, (NB - 3) // 2, body, 0)
        fire(xi, NB - 1, 0, False)
        process(xi, 1)
        process(xi, 0)
        drain_scat(1)
        drain_scat(0)

        @pl.when(s < NTAIL)
        def _():
            blk = cbase + 2 * NB
            off = blk * CHUNK
            pltpu.sync_copy(cols1.at[pl.ds(off, CHUNK)], cb0.at[pl.ds(0, CHUNK)])
            for g in range(CHUNK // 16):
                cb0[pl.ds(g * 16, 16)] = cb0[pl.ds(g * 16, 16)] + coff
            pltpu.sync_copy(rows3.at[pl.ds(blk, 1)], rb0.at[pl.ds(0, 1)])
            pltpu.sync_copy(vals1.at[pl.ds(off, CHUNK)], vb0.at[pl.ds(0, CHUNK)])
            pltpu.async_copy(
                xi.at[cb0.at[pl.ds(0, CHUNK)]], ga0.at[pl.ds(0, CHUNK)], gsem0).wait()
            scale(ga0, vb0, CHUNK)
            pltpu.sync_copy(ga0.at[pl.ds(0, CHUNK)], acc.at[rb0.at[0]], add=True)

        plsc.subcore_barrier()

        for k in range(NWB):
            b = row0 + k * WB
            pltpu.sync_copy(acc.at[pl.ds(b, WB)], stage)
            pltpu.sync_copy(stage, xo.at[pl.ds(c * NODES + b, WB)])

    ga0v = ga0.at[pl.ds(0, WB)]
    ga1v = ga1.at[pl.ds(0, WB)]
    for k in range(NWB):
        b = c * NODES + row0 + k * WB
        if k > 0:
            pltpu.make_async_copy(
                stage, outm.at[pl.ds(b - WB, WB)], ssem0).wait()
        pltpu.async_copy(x0.at[pl.ds(b, WB)], stage, gsem0)
        pltpu.async_copy(x1.at[pl.ds(b, WB)], zer, gsem1)
        pltpu.async_copy(x2.at[pl.ds(b, WB)], ga0v, ssem1)
        pltpu.async_copy(x3.at[pl.ds(b, WB)], ga1v, msem)
        pltpu.make_async_copy(x0.at[pl.ds(b, WB)], stage, gsem0).wait()
        pltpu.make_async_copy(x1.at[pl.ds(b, WB)], zer, gsem1).wait()
        pltpu.make_async_copy(x2.at[pl.ds(b, WB)], ga0v, ssem1).wait()
        pltpu.make_async_copy(x3.at[pl.ds(b, WB)], ga1v, msem).wait()

        def addb(r, _):
            q = jnp.float32(0.25)
            lo = (stage[r, pl.ds(0, 16)] + zer[r, pl.ds(0, 16)]
                  + ga0[r, pl.ds(0, 16)] + ga1[r, pl.ds(0, 16)]) * q
            hi = (stage[r, pl.ds(16, 16)] + zer[r, pl.ds(16, 16)]
                  + ga0[r, pl.ds(16, 16)] + ga1[r, pl.ds(16, 16)]) * q
            stage[r, pl.ds(0, 16)] = lo
            stage[r, pl.ds(16, 16)] = hi
            return 0
        lax.fori_loop(0, WB, addb, 0)
        pltpu.async_copy(stage, outm.at[pl.ds(b, WB)], ssem0)
    pltpu.make_async_copy(
        stage, outm.at[pl.ds(c * NODES + row0 + (NWB - 1) * WB, WB)], ssem0).wait()


@jax.jit
def _lgcn_sc(x0, cols1, rows3, vals1):
    mesh = plsc.VectorSubcoreMesh(core_axis_name="c", subcore_axis_name="s")
    f = pl.kernel(
        _sc_body,
        out_type=[
            jax.ShapeDtypeStruct((2 * NODES, HALF), jnp.float32),
            jax.ShapeDtypeStruct((2 * NODES, HALF), jnp.float32),
            jax.ShapeDtypeStruct((2 * NODES, HALF), jnp.float32),
            jax.ShapeDtypeStruct((2 * NODES, HALF), jnp.float32),
        ],
        mesh=mesh,
        scratch_types=[
            pltpu.VMEM_SHARED((NODES, HALF), jnp.float32),
            pltpu.VMEM((BLK, HALF), jnp.float32),
            pltpu.VMEM((BLK, HALF), jnp.float32),
            pltpu.VMEM((BLK,), jnp.int32),
            pltpu.VMEM((BLK,), jnp.int32),
            pltpu.VMEM((2, CHUNK), jnp.int32),
            pltpu.VMEM((2, CHUNK), jnp.int32),
            pltpu.VMEM((BLK,), jnp.float32),
            pltpu.VMEM((BLK,), jnp.float32),
            pltpu.VMEM((WB, HALF), jnp.float32),
            pltpu.VMEM((WB, HALF), jnp.float32),
            pltpu.SemaphoreType.DMA,
            pltpu.SemaphoreType.DMA,
            pltpu.SemaphoreType.DMA,
            pltpu.SemaphoreType.DMA,
            pltpu.SemaphoreType.DMA,
        ],
        compiler_params=pltpu.CompilerParams(
            use_tc_tiling_on_sc=False, needs_layout_passes=False),
    )
    return f(x0, cols1, rows3, vals1)


def kernel(user_emb, item_emb, user_prototypes, item_prototypes, adj_indices, adj_values):
    x0 = jnp.concatenate(
        [user_emb[:, :HALF], item_emb[:, :HALF],
         user_emb[:, HALF:], item_emb[:, HALF:]], axis=0)
    cols1 = adj_indices[1]
    rows3 = adj_indices[0].reshape(NCHT, CHUNK)
    vals1 = adj_values

    outm = _lgcn_sc(x0, cols1, rows3, vals1)[0]

    lgcn = jnp.concatenate([outm[:NODES], outm[NODES:]], axis=1)
    user_all = lgcn[:USER_N]
    item_all = lgcn[USER_N:]
    return (user_all, item_all, user_emb, item_emb,
            user_prototypes, item_prototypes)


# speedup vs baseline: 9.7535x; 1.4294x over previous
import jax
import jax.numpy as jnp
from jax import lax
from jax.experimental import pallas as pl
from jax.experimental.pallas import tpu as pltpu
from jax.experimental.pallas import tpu_sc as plsc

USER_N = 25000
ITEM_N = 25000
NODES = USER_N + ITEM_N
EMB = 64
HALF = EMB // 2
LAYERS = 3
EDGES = 800000
NC = 2
NS = 16
CHUNK = 128
BLK = 2 * CHUNK
NCHT = EDGES // CHUNK
NB = 195
NTAIL = NCHT - NS * 2 * NB
STRIPE = NODES // NS
WB = 125
NWB = STRIPE // WB


def _zero2d(ref, nrows):
    def body(r, _):
        ref[r, pl.ds(0, 16)] = jnp.zeros((16,), jnp.float32)
        ref[r, pl.ds(16, 16)] = jnp.zeros((16,), jnp.float32)
        return 0
    lax.fori_loop(0, nrows, body, 0)


def _sc_body(x0, cols1, rows3, vals1, outm, x1, x2, x3,
             acc, ga0, ga1, cb0, cb1, rb0, rb1, vb0, vb1,
             zer, stage, gsem0, gsem1, ssem0, ssem1, msem, isem0, isem1):
    c = lax.axis_index("c")
    s = lax.axis_index("s")
    row0 = s * STRIPE
    cbase = s * (2 * NB) + jnp.minimum(s, NTAIL)
    coff = jnp.full((16,), c * NODES, jnp.int32)

    _zero2d(zer, WB)

    gas = (ga0, ga1)
    cbs = (cb0, cb1)
    rbs = (rb0, rb1)
    vbs = (vb0, vb1)
    gsems = (gsem0, gsem1)
    ssems = (ssem0, ssem1)
    isems = (isem0, isem1)

    def drain_scat(p):
        pltpu.make_async_copy(
            gas[p].at[pl.ds(0, CHUNK)], acc.at[rbs[p].at[0]], ssems[p]).wait()
        pltpu.make_async_copy(
            gas[p].at[pl.ds(CHUNK, CHUNK)], acc.at[rbs[p].at[1]], ssems[p]).wait()

    def fire(xi, j, p, first):
        if not first:
            drain_scat(p)
        blk = cbase + 2 * j
        off = blk * CHUNK
        if first:
            pltpu.sync_copy(cols1.at[pl.ds(off, BLK)], cbs[p])
            pltpu.sync_copy(vals1.at[pl.ds(off, BLK)], vbs[p])
        else:
            pltpu.make_async_copy(cols1.at[pl.ds(off, BLK)], cbs[p], isems[p]).wait()
            pltpu.make_async_copy(vals1.at[pl.ds(off, BLK)], vbs[p], isems[p]).wait()
        for g in range(BLK // 16):
            cbs[p][pl.ds(g * 16, 16)] = cbs[p][pl.ds(g * 16, 16)] + coff
        pltpu.sync_copy(rows3.at[pl.ds(blk, 2)], rbs[p])
        pltpu.async_copy(
            xi.at[cbs[p].at[pl.ds(0, CHUNK)]], gas[p].at[pl.ds(0, CHUNK)], gsems[p])
        pltpu.async_copy(
            xi.at[cbs[p].at[pl.ds(CHUNK, CHUNK)]], gas[p].at[pl.ds(CHUNK, CHUNK)], gsems[p])

    def scale(ga, vb, nedge):
        @plsc.parallel_loop(0, nedge // 16, step=1, unroll=2)
        def grp(g):
            vv = vb[pl.ds(g * 16, 16)]
            for e in range(16):
                j = g * 16 + e
                bcast = vv[jnp.full((16,), e, jnp.int32)]
                ga[j, pl.ds(0, 16)] = ga[j, pl.ds(0, 16)] * bcast
                ga[j, pl.ds(16, 16)] = ga[j, pl.ds(16, 16)] * bcast

    def process(xi, p, pf_j=None):
        pltpu.make_async_copy(
            xi.at[cbs[p].at[pl.ds(0, CHUNK)]], gas[p].at[pl.ds(0, CHUNK)], gsems[p]).wait()
        pltpu.make_async_copy(
            xi.at[cbs[p].at[pl.ds(CHUNK, CHUNK)]], gas[p].at[pl.ds(CHUNK, CHUNK)], gsems[p]).wait()
        if pf_j is not None:
            pfoff = (cbase + 2 * pf_j) * CHUNK
            pltpu.async_copy(cols1.at[pl.ds(pfoff, BLK)], cbs[p], isems[p])
        scale(gas[p], vbs[p], BLK)
        if pf_j is not None:
            pfoff = (cbase + 2 * pf_j) * CHUNK
            pltpu.async_copy(vals1.at[pl.ds(pfoff, BLK)], vbs[p], isems[p])
        pltpu.async_copy(
            gas[p].at[pl.ds(0, CHUNK)], acc.at[rbs[p].at[0]], ssems[p], add=True)
        pltpu.async_copy(
            gas[p].at[pl.ds(CHUNK, CHUNK)], acc.at[rbs[p].at[1]], ssems[p], add=True)

    xs_in = (x0, x1, x2)
    xs_out = (x1, x2, x3)
    for l in range(LAYERS):
        xi = xs_in[l]
        xo = xs_out[l]
        for k in range(NWB):
            pltpu.async_copy(zer, acc.at[pl.ds(row0 + k * WB, WB)], gsem0)
        for k in range(NWB):
            pltpu.make_async_copy(zer, acc.at[pl.ds(row0 + k * WB, WB)], gsem0).wait()
        plsc.subcore_barrier()

        fire(xi, 0, 0, True)
        fire(xi, 1, 1, True)
        process(xi, 0, 2)

        def body(i2, _):
            j = 2 * i2
            fire(xi, j + 2, 0, False)
            process(xi, 1, j + 3)
            fire(xi, j + 3, 1, False)
            process(xi, 0, j + 4)
            return 0
        lax.fori_loop(0, (NB - 3) // 2, body, 0)
        fire(xi, NB - 1, 0, False)
        process(xi, 1)
        process(xi, 0)
        drain_scat(1)
        drain_scat(0)

        @pl.when(s < NTAIL)
        def _():
            blk = cbase + 2 * NB
            off = blk * CHUNK
            pltpu.sync_copy(cols1.at[pl.ds(off, CHUNK)], cb0.at[pl.ds(0, CHUNK)])
            for g in range(CHUNK // 16):
                cb0[pl.ds(g * 16, 16)] = cb0[pl.ds(g * 16, 16)] + coff
            pltpu.sync_copy(rows3.at[pl.ds(blk, 1)], rb0.at[pl.ds(0, 1)])
            pltpu.sync_copy(vals1.at[pl.ds(off, CHUNK)], vb0.at[pl.ds(0, CHUNK)])
            pltpu.async_copy(
                xi.at[cb0.at[pl.ds(0, CHUNK)]], ga0.at[pl.ds(0, CHUNK)], gsem0).wait()
            scale(ga0, vb0, CHUNK)
            pltpu.sync_copy(ga0.at[pl.ds(0, CHUNK)], acc.at[rb0.at[0]], add=True)

        plsc.subcore_barrier()

        for k in range(NWB):
            b = row0 + k * WB
            pltpu.sync_copy(acc.at[pl.ds(b, WB)], stage)
            pltpu.sync_copy(stage, xo.at[pl.ds(c * NODES + b, WB)])

    ga0v = ga0.at[pl.ds(0, WB)]
    ga1v = ga1.at[pl.ds(0, WB)]
    for k in range(NWB):
        b = c * NODES + row0 + k * WB
        if k > 0:
            pltpu.make_async_copy(
                stage, outm.at[pl.ds(b - WB, WB)], ssem0).wait()
        pltpu.async_copy(x0.at[pl.ds(b, WB)], stage, gsem0)
        pltpu.async_copy(x1.at[pl.ds(b, WB)], zer, gsem1)
        pltpu.async_copy(x2.at[pl.ds(b, WB)], ga0v, ssem1)
        pltpu.async_copy(x3.at[pl.ds(b, WB)], ga1v, msem)
        pltpu.make_async_copy(x0.at[pl.ds(b, WB)], stage, gsem0).wait()
        pltpu.make_async_copy(x1.at[pl.ds(b, WB)], zer, gsem1).wait()
        pltpu.make_async_copy(x2.at[pl.ds(b, WB)], ga0v, ssem1).wait()
        pltpu.make_async_copy(x3.at[pl.ds(b, WB)], ga1v, msem).wait()

        def addb(r, _):
            q = jnp.float32(0.25)
            lo = (stage[r, pl.ds(0, 16)] + zer[r, pl.ds(0, 16)]
                  + ga0[r, pl.ds(0, 16)] + ga1[r, pl.ds(0, 16)]) * q
            hi = (stage[r, pl.ds(16, 16)] + zer[r, pl.ds(16, 16)]
                  + ga0[r, pl.ds(16, 16)] + ga1[r, pl.ds(16, 16)]) * q
            stage[r, pl.ds(0, 16)] = lo
            stage[r, pl.ds(16, 16)] = hi
            return 0
        lax.fori_loop(0, WB, addb, 0)
        pltpu.async_copy(stage, outm.at[pl.ds(b, WB)], ssem0)
    pltpu.make_async_copy(
        stage, outm.at[pl.ds(c * NODES + row0 + (NWB - 1) * WB, WB)], ssem0).wait()


@jax.jit
def _lgcn_sc(x0, cols1, rows3, vals1):
    mesh = plsc.VectorSubcoreMesh(core_axis_name="c", subcore_axis_name="s")
    f = pl.kernel(
        _sc_body,
        out_type=[
            jax.ShapeDtypeStruct((2 * NODES, HALF), jnp.float32),
            jax.ShapeDtypeStruct((2 * NODES, HALF), jnp.float32),
            jax.ShapeDtypeStruct((2 * NODES, HALF), jnp.float32),
            jax.ShapeDtypeStruct((2 * NODES, HALF), jnp.float32),
        ],
        mesh=mesh,
        scratch_types=[
            pltpu.VMEM_SHARED((NODES, HALF), jnp.float32),
            pltpu.VMEM((BLK, HALF), jnp.float32),
            pltpu.VMEM((BLK, HALF), jnp.float32),
            pltpu.VMEM((BLK,), jnp.int32),
            pltpu.VMEM((BLK,), jnp.int32),
            pltpu.VMEM((2, CHUNK), jnp.int32),
            pltpu.VMEM((2, CHUNK), jnp.int32),
            pltpu.VMEM((BLK,), jnp.float32),
            pltpu.VMEM((BLK,), jnp.float32),
            pltpu.VMEM((WB, HALF), jnp.float32),
            pltpu.VMEM((WB, HALF), jnp.float32),
            pltpu.SemaphoreType.DMA,
            pltpu.SemaphoreType.DMA,
            pltpu.SemaphoreType.DMA,
            pltpu.SemaphoreType.DMA,
            pltpu.SemaphoreType.DMA,
            pltpu.SemaphoreType.DMA,
            pltpu.SemaphoreType.DMA,
        ],
        compiler_params=pltpu.CompilerParams(
            use_tc_tiling_on_sc=False, needs_layout_passes=False),
    )
    return f(x0, cols1, rows3, vals1)


def kernel(user_emb, item_emb, user_prototypes, item_prototypes, adj_indices, adj_values):
    x0 = jnp.concatenate(
        [user_emb[:, :HALF], item_emb[:, :HALF],
         user_emb[:, HALF:], item_emb[:, HALF:]], axis=0)
    cols1 = adj_indices[1]
    rows3 = adj_indices[0].reshape(NCHT, CHUNK)
    vals1 = adj_values

    outm = _lgcn_sc(x0, cols1, rows3, vals1)[0]

    lgcn = jnp.concatenate([outm[:NODES], outm[NODES:]], axis=1)
    user_all = lgcn[:USER_N]
    item_all = lgcn[USER_N:]
    return (user_all, item_all, user_emb, item_emb,
            user_prototypes, item_prototypes)


# speedup vs baseline: 10.8865x; 1.1162x over previous
import jax
import jax.numpy as jnp
from jax import lax
from jax.experimental import pallas as pl
from jax.experimental.pallas import tpu as pltpu
from jax.experimental.pallas import tpu_sc as plsc

USER_N = 25000
ITEM_N = 25000
NODES = USER_N + ITEM_N
EMB = 64
HALF = EMB // 2
LAYERS = 3
EDGES = 800000
NC = 2
NS = 16
CHUNK = 128
BLK = 2 * CHUNK
NCHT = EDGES // CHUNK
NB = 195
NTAIL = NCHT - NS * 2 * NB
STRIPE = NODES // NS
WB = 125
NWB = STRIPE // WB


def _zero2d(ref, nrows):
    def body(r, _):
        ref[r, pl.ds(0, 16)] = jnp.zeros((16,), jnp.float32)
        ref[r, pl.ds(16, 16)] = jnp.zeros((16,), jnp.float32)
        return 0
    lax.fori_loop(0, nrows, body, 0)


def _sc_body(user_e, item_e, cols1, rows3, vals1, outu, outi, x0b, x1, x2, x3,
             acc, ga0, ga1, cb0, cb1, rb0, rb1, vb0, vb1,
             zer, stage, gsem0, gsem1, ssem0, ssem1, msem, isem0, isem1):
    c = lax.axis_index("c")
    s = lax.axis_index("s")
    row0 = s * STRIPE
    cbase = s * (2 * NB) + jnp.minimum(s, NTAIL)
    coff = jnp.full((16,), c * NODES, jnp.int32)

    _zero2d(zer, WB)

    gas = (ga0, ga1)
    cbs = (cb0, cb1)
    rbs = (rb0, rb1)
    vbs = (vb0, vb1)
    gsems = (gsem0, gsem1)
    ssems = (ssem0, ssem1)
    isems = (isem0, isem1)

    def drain_scat(p):
        pltpu.make_async_copy(
            gas[p].at[pl.ds(0, CHUNK)], acc.at[rbs[p].at[0]], ssems[p]).wait()
        pltpu.make_async_copy(
            gas[p].at[pl.ds(CHUNK, CHUNK)], acc.at[rbs[p].at[1]], ssems[p]).wait()

    def fire(xi, j, p, first):
        if not first:
            drain_scat(p)
        blk = cbase + 2 * j
        off = blk * CHUNK
        if first:
            pltpu.sync_copy(cols1.at[pl.ds(off, BLK)], cbs[p])
            pltpu.sync_copy(vals1.at[pl.ds(off, BLK)], vbs[p])
        else:
            pltpu.make_async_copy(cols1.at[pl.ds(off, BLK)], cbs[p], isems[p]).wait()
            pltpu.make_async_copy(vals1.at[pl.ds(off, BLK)], vbs[p], isems[p]).wait()
        for g in range(BLK // 16):
            cbs[p][pl.ds(g * 16, 16)] = cbs[p][pl.ds(g * 16, 16)] + coff
        pltpu.sync_copy(rows3.at[pl.ds(blk, 2)], rbs[p])
        pltpu.async_copy(
            xi.at[cbs[p].at[pl.ds(0, CHUNK)]], gas[p].at[pl.ds(0, CHUNK)], gsems[p])
        pltpu.async_copy(
            xi.at[cbs[p].at[pl.ds(CHUNK, CHUNK)]], gas[p].at[pl.ds(CHUNK, CHUNK)], gsems[p])

    def scale(ga, vb, nedge):
        @plsc.parallel_loop(0, nedge // 16, step=1, unroll=2)
        def grp(g):
            vv = vb[pl.ds(g * 16, 16)]
            for e in range(16):
                j = g * 16 + e
                bcast = vv[jnp.full((16,), e, jnp.int32)]
                ga[j, pl.ds(0, 16)] = ga[j, pl.ds(0, 16)] * bcast
                ga[j, pl.ds(16, 16)] = ga[j, pl.ds(16, 16)] * bcast

    def process(xi, p, pf_j=None):
        pltpu.make_async_copy(
            xi.at[cbs[p].at[pl.ds(0, CHUNK)]], gas[p].at[pl.ds(0, CHUNK)], gsems[p]).wait()
        pltpu.make_async_copy(
            xi.at[cbs[p].at[pl.ds(CHUNK, CHUNK)]], gas[p].at[pl.ds(CHUNK, CHUNK)], gsems[p]).wait()
        if pf_j is not None:
            pfoff = (cbase + 2 * pf_j) * CHUNK
            pltpu.async_copy(cols1.at[pl.ds(pfoff, BLK)], cbs[p], isems[p])
        scale(gas[p], vbs[p], BLK)
        if pf_j is not None:
            pfoff = (cbase + 2 * pf_j) * CHUNK
            pltpu.async_copy(vals1.at[pl.ds(pfoff, BLK)], vbs[p], isems[p])
        pltpu.async_copy(
            gas[p].at[pl.ds(0, CHUNK)], acc.at[rbs[p].at[0]], ssems[p], add=True)
        pltpu.async_copy(
            gas[p].at[pl.ds(CHUNK, CHUNK)], acc.at[rbs[p].at[1]], ssems[p], add=True)

    ga0v = ga0.at[pl.ds(0, WB)]
    bufs = (stage, ga0v)
    half = NS // 2
    for k in range(NWB):
        buf = bufs[k % 2]
        n = row0 + k * WB
        if k >= 2:
            pn = row0 + (k - 2) * WB
            pltpu.make_async_copy(
                bufs[k % 2], x0b.at[pl.ds(c * NODES + pn, WB)], ssem0).wait()

        @pl.when(s < half)
        def _():
            pltpu.sync_copy(user_e.at[pl.ds(n, WB), pl.ds(c * HALF, HALF)], buf)

        @pl.when(s >= half)
        def _():
            pltpu.sync_copy(item_e.at[pl.ds(n - USER_N, WB), pl.ds(c * HALF, HALF)], buf)
        pltpu.async_copy(buf, x0b.at[pl.ds(c * NODES + n, WB)], ssem0)
    for k in (NWB - 2, NWB - 1):
        pltpu.make_async_copy(
            bufs[k % 2], x0b.at[pl.ds(c * NODES + row0 + k * WB, WB)], ssem0).wait()
    plsc.subcore_barrier()

    xs_in = (x0b, x1, x2)
    xs_out = (x1, x2, x3)
    for l in range(LAYERS):
        xi = xs_in[l]
        xo = xs_out[l]
        for k in range(NWB):
            pltpu.async_copy(zer, acc.at[pl.ds(row0 + k * WB, WB)], gsem0)
        for k in range(NWB):
            pltpu.make_async_copy(zer, acc.at[pl.ds(row0 + k * WB, WB)], gsem0).wait()
        plsc.subcore_barrier()

        fire(xi, 0, 0, True)
        fire(xi, 1, 1, True)
        process(xi, 0, 2)

        def body(i2, _):
            j = 2 * i2
            fire(xi, j + 2, 0, False)
            process(xi, 1, j + 3)
            fire(xi, j + 3, 1, False)
            process(xi, 0, j + 4)
            return 0
        lax.fori_loop(0, (NB - 3) // 2, body, 0)
        fire(xi, NB - 1, 0, False)
        process(xi, 1)
        process(xi, 0)
        drain_scat(1)
        drain_scat(0)

        @pl.when(s < NTAIL)
        def _():
            blk = cbase + 2 * NB
            off = blk * CHUNK
            pltpu.sync_copy(cols1.at[pl.ds(off, CHUNK)], cb0.at[pl.ds(0, CHUNK)])
            for g in range(CHUNK // 16):
                cb0[pl.ds(g * 16, 16)] = cb0[pl.ds(g * 16, 16)] + coff
            pltpu.sync_copy(rows3.at[pl.ds(blk, 1)], rb0.at[pl.ds(0, 1)])
            pltpu.sync_copy(vals1.at[pl.ds(off, CHUNK)], vb0.at[pl.ds(0, CHUNK)])
            pltpu.async_copy(
                xi.at[cb0.at[pl.ds(0, CHUNK)]], ga0.at[pl.ds(0, CHUNK)], gsem0).wait()
            scale(ga0, vb0, CHUNK)
            pltpu.sync_copy(ga0.at[pl.ds(0, CHUNK)], acc.at[rb0.at[0]], add=True)

        plsc.subcore_barrier()

        for k in range(NWB):
            b = row0 + k * WB
            pltpu.sync_copy(acc.at[pl.ds(b, WB)], stage)
            pltpu.sync_copy(stage, xo.at[pl.ds(c * NODES + b, WB)])

    ga1v = ga1.at[pl.ds(0, WB)]
    for k in range(NWB):
        b = c * NODES + row0 + k * WB
        pltpu.async_copy(x0b.at[pl.ds(b, WB)], stage, gsem0)
        pltpu.async_copy(x1.at[pl.ds(b, WB)], zer, gsem1)
        pltpu.async_copy(x2.at[pl.ds(b, WB)], ga0v, ssem1)
        pltpu.async_copy(x3.at[pl.ds(b, WB)], ga1v, msem)
        pltpu.make_async_copy(x0b.at[pl.ds(b, WB)], stage, gsem0).wait()
        pltpu.make_async_copy(x1.at[pl.ds(b, WB)], zer, gsem1).wait()
        pltpu.make_async_copy(x2.at[pl.ds(b, WB)], ga0v, ssem1).wait()
        pltpu.make_async_copy(x3.at[pl.ds(b, WB)], ga1v, msem).wait()

        def addb(r, _):
            q = jnp.float32(0.25)
            lo = (stage[r, pl.ds(0, 16)] + zer[r, pl.ds(0, 16)]
                  + ga0[r, pl.ds(0, 16)] + ga1[r, pl.ds(0, 16)]) * q
            hi = (stage[r, pl.ds(16, 16)] + zer[r, pl.ds(16, 16)]
                  + ga0[r, pl.ds(16, 16)] + ga1[r, pl.ds(16, 16)]) * q
            stage[r, pl.ds(0, 16)] = lo
            stage[r, pl.ds(16, 16)] = hi
            return 0
        lax.fori_loop(0, WB, addb, 0)

        @pl.when(s < NS // 2)
        def _():
            pltpu.sync_copy(
                stage, outu.at[pl.ds(row0 + k * WB, WB), pl.ds(c * HALF, HALF)])

        @pl.when(s >= NS // 2)
        def _():
            pltpu.sync_copy(
                stage, outi.at[pl.ds(row0 - USER_N + k * WB, WB), pl.ds(c * HALF, HALF)])


@jax.jit
def _lgcn_sc(user_e, item_e, cols1, rows3, vals1):
    mesh = plsc.VectorSubcoreMesh(core_axis_name="c", subcore_axis_name="s")
    f = pl.kernel(
        _sc_body,
        out_type=[
            jax.ShapeDtypeStruct((USER_N, EMB), jnp.float32),
            jax.ShapeDtypeStruct((ITEM_N, EMB), jnp.float32),
            jax.ShapeDtypeStruct((2 * NODES, HALF), jnp.float32),
            jax.ShapeDtypeStruct((2 * NODES, HALF), jnp.float32),
            jax.ShapeDtypeStruct((2 * NODES, HALF), jnp.float32),
            jax.ShapeDtypeStruct((2 * NODES, HALF), jnp.float32),
        ],
        mesh=mesh,
        scratch_types=[
            pltpu.VMEM_SHARED((NODES, HALF), jnp.float32),
            pltpu.VMEM((BLK, HALF), jnp.float32),
            pltpu.VMEM((BLK, HALF), jnp.float32),
            pltpu.VMEM((BLK,), jnp.int32),
            pltpu.VMEM((BLK,), jnp.int32),
            pltpu.VMEM((2, CHUNK), jnp.int32),
            pltpu.VMEM((2, CHUNK), jnp.int32),
            pltpu.VMEM((BLK,), jnp.float32),
            pltpu.VMEM((BLK,), jnp.float32),
            pltpu.VMEM((WB, HALF), jnp.float32),
            pltpu.VMEM((WB, HALF), jnp.float32),
            pltpu.SemaphoreType.DMA,
            pltpu.SemaphoreType.DMA,
            pltpu.SemaphoreType.DMA,
            pltpu.SemaphoreType.DMA,
            pltpu.SemaphoreType.DMA,
            pltpu.SemaphoreType.DMA,
            pltpu.SemaphoreType.DMA,
        ],
        compiler_params=pltpu.CompilerParams(
            use_tc_tiling_on_sc=False, needs_layout_passes=False),
    )
    return f(user_e, item_e, cols1, rows3, vals1)


def kernel(user_emb, item_emb, user_prototypes, item_prototypes, adj_indices, adj_values):
    cols1 = adj_indices[1]
    rows3 = adj_indices[0].reshape(NCHT, CHUNK)
    outs = _lgcn_sc(user_emb, item_emb, cols1, rows3, adj_values)
    return (outs[0], outs[1], user_emb, item_emb,
            user_prototypes, item_prototypes)


# speedup vs baseline: 11.0903x; 1.0187x over previous
import jax
import jax.numpy as jnp
from jax import lax
from jax.experimental import pallas as pl
from jax.experimental.pallas import tpu as pltpu
from jax.experimental.pallas import tpu_sc as plsc

USER_N = 25000
ITEM_N = 25000
NODES = USER_N + ITEM_N
EMB = 64
HALF = EMB // 2
LAYERS = 3
EDGES = 800000
NC = 2
NS = 16
CHUNK = 128
BLK = 2 * CHUNK
NCHT = EDGES // CHUNK
NB = 195
NTAIL = NCHT - NS * 2 * NB
STRIPE = NODES // NS
WB = 125
NWB = STRIPE // WB


def _zero2d(ref, nrows):
    def body(r, _):
        ref[r, pl.ds(0, 16)] = jnp.zeros((16,), jnp.float32)
        ref[r, pl.ds(16, 16)] = jnp.zeros((16,), jnp.float32)
        return 0
    lax.fori_loop(0, nrows, body, 0)


def _sc_body(user_e, item_e, cols1, rows1, vals1, outu, outi, x0b, x1, x2, x3,
             acc, ga0, ga1, cb0, cb1, rb0, rb1, rbt, vb0, vb1,
             zer, stage, gsem0, gsem1, ssem0, ssem1, msem, isem0, isem1):
    c = lax.axis_index("c")
    s = lax.axis_index("s")
    row0 = s * STRIPE
    cbase = s * (2 * NB) + jnp.minimum(s, NTAIL)
    coff = jnp.full((16,), c * NODES, jnp.int32)

    _zero2d(zer, WB)

    gas = (ga0, ga1)
    cbs = (cb0, cb1)
    rbs = (rb0, rb1)
    vbs = (vb0, vb1)
    gsems = (gsem0, gsem1)
    ssems = (ssem0, ssem1)
    isems = (isem0, isem1)

    def drain_scat(p):
        pltpu.make_async_copy(gas[p], acc.at[rbs[p]], ssems[p]).wait()

    def fire(xi, j, p, first):
        if not first:
            drain_scat(p)
        blk = cbase + 2 * j
        off = blk * CHUNK
        if first:
            pltpu.sync_copy(cols1.at[pl.ds(off, BLK)], cbs[p])
            pltpu.sync_copy(vals1.at[pl.ds(off, BLK)], vbs[p])
        else:
            pltpu.make_async_copy(cols1.at[pl.ds(off, BLK)], cbs[p], isems[p]).wait()
            pltpu.make_async_copy(vals1.at[pl.ds(off, BLK)], vbs[p], isems[p]).wait()
        for g in range(BLK // 16):
            cbs[p][pl.ds(g * 16, 16)] = cbs[p][pl.ds(g * 16, 16)] + coff
        pltpu.sync_copy(rows1.at[pl.ds(off, BLK)], rbs[p])
        pltpu.async_copy(xi.at[cbs[p]], gas[p], gsems[p])

    def scale(ga, vb, nedge):
        @plsc.parallel_loop(0, nedge // 16, step=1, unroll=2)
        def grp(g):
            vv = vb[pl.ds(g * 16, 16)]
            for e in range(16):
                j = g * 16 + e
                bcast = vv[jnp.full((16,), e, jnp.int32)]
                ga[j, pl.ds(0, 16)] = ga[j, pl.ds(0, 16)] * bcast
                ga[j, pl.ds(16, 16)] = ga[j, pl.ds(16, 16)] * bcast

    def process(xi, p, pf_j=None):
        pltpu.make_async_copy(xi.at[cbs[p]], gas[p], gsems[p]).wait()
        if pf_j is not None:
            pfoff = (cbase + 2 * pf_j) * CHUNK
            pltpu.async_copy(cols1.at[pl.ds(pfoff, BLK)], cbs[p], isems[p])
        scale(gas[p], vbs[p], BLK)
        if pf_j is not None:
            pfoff = (cbase + 2 * pf_j) * CHUNK
            pltpu.async_copy(vals1.at[pl.ds(pfoff, BLK)], vbs[p], isems[p])
        pltpu.async_copy(gas[p], acc.at[rbs[p]], ssems[p], add=True)

    ga0v = ga0.at[pl.ds(0, WB)]
    bufs = (stage, ga0v)
    half = NS // 2
    for k in range(NWB):
        buf = bufs[k % 2]
        n = row0 + k * WB
        if k >= 2:
            pn = row0 + (k - 2) * WB
            pltpu.make_async_copy(
                bufs[k % 2], x0b.at[pl.ds(c * NODES + pn, WB)], ssem0).wait()

        @pl.when(s < half)
        def _():
            pltpu.sync_copy(user_e.at[pl.ds(n, WB), pl.ds(c * HALF, HALF)], buf)

        @pl.when(s >= half)
        def _():
            pltpu.sync_copy(item_e.at[pl.ds(n - USER_N, WB), pl.ds(c * HALF, HALF)], buf)
        pltpu.async_copy(buf, x0b.at[pl.ds(c * NODES + n, WB)], ssem0)
        pltpu.async_copy(zer, acc.at[pl.ds(n, WB)], gsem0)
    for k in range(NWB):
        pltpu.make_async_copy(zer, acc.at[pl.ds(row0 + k * WB, WB)], gsem0).wait()
    for k in (NWB - 2, NWB - 1):
        pltpu.make_async_copy(
            bufs[k % 2], x0b.at[pl.ds(c * NODES + row0 + k * WB, WB)], ssem0).wait()
    plsc.subcore_barrier()

    xs_in = (x0b, x1, x2)
    xs_out = (x1, x2, x3)
    for l in range(LAYERS):
        xi = xs_in[l]
        xo = xs_out[l]
        fire(xi, 0, 0, True)
        fire(xi, 1, 1, True)
        process(xi, 0, 2)

        def body(i2, _):
            j = 2 * i2
            fire(xi, j + 2, 0, False)
            process(xi, 1, j + 3)
            fire(xi, j + 3, 1, False)
            process(xi, 0, j + 4)
            return 0
        lax.fori_loop(0, (NB - 3) // 2, body, 0)
        fire(xi, NB - 1, 0, False)
        process(xi, 1)
        process(xi, 0)
        drain_scat(1)
        drain_scat(0)

        @pl.when(s < NTAIL)
        def _():
            blk = cbase + 2 * NB
            off = blk * CHUNK
            pltpu.sync_copy(cols1.at[pl.ds(off, CHUNK)], cb0.at[pl.ds(0, CHUNK)])
            for g in range(CHUNK // 16):
                cb0[pl.ds(g * 16, 16)] = cb0[pl.ds(g * 16, 16)] + coff
            pltpu.sync_copy(rows1.at[pl.ds(off, CHUNK)], rbt)
            pltpu.sync_copy(vals1.at[pl.ds(off, CHUNK)], vb0.at[pl.ds(0, CHUNK)])
            pltpu.async_copy(
                xi.at[cb0.at[pl.ds(0, CHUNK)]], ga0.at[pl.ds(0, CHUNK)], gsem0).wait()
            scale(ga0, vb0, CHUNK)
            pltpu.sync_copy(ga0.at[pl.ds(0, CHUNK)], acc.at[rbt], add=True)

        plsc.subcore_barrier()

        for k in range(NWB):
            buf = bufs[k % 2]
            b = row0 + k * WB
            if k >= 2:
                pltpu.make_async_copy(
                    buf, xo.at[pl.ds(c * NODES + b - 2 * WB, WB)], ssem0).wait()
            pltpu.sync_copy(acc.at[pl.ds(b, WB)], buf)
            if l < LAYERS - 1:
                pltpu.async_copy(zer, acc.at[pl.ds(b, WB)], gsem0)
            pltpu.async_copy(buf, xo.at[pl.ds(c * NODES + b, WB)], ssem0)
        for k in (NWB - 2, NWB - 1):
            pltpu.make_async_copy(
                bufs[k % 2], xo.at[pl.ds(c * NODES + row0 + k * WB, WB)], ssem0).wait()
        if l < LAYERS - 1:
            for k in range(NWB):
                pltpu.make_async_copy(
                    zer, acc.at[pl.ds(row0 + k * WB, WB)], gsem0).wait()
        plsc.subcore_barrier()

    ga1v = ga1.at[pl.ds(0, WB)]
    for k in range(NWB):
        b = c * NODES + row0 + k * WB
        pltpu.async_copy(x0b.at[pl.ds(b, WB)], stage, gsem0)
        pltpu.async_copy(x1.at[pl.ds(b, WB)], zer, gsem1)
        pltpu.async_copy(x2.at[pl.ds(b, WB)], ga0v, ssem1)
        pltpu.async_copy(x3.at[pl.ds(b, WB)], ga1v, msem)
        pltpu.make_async_copy(x0b.at[pl.ds(b, WB)], stage, gsem0).wait()
        pltpu.make_async_copy(x1.at[pl.ds(b, WB)], zer, gsem1).wait()
        pltpu.make_async_copy(x2.at[pl.ds(b, WB)], ga0v, ssem1).wait()
        pltpu.make_async_copy(x3.at[pl.ds(b, WB)], ga1v, msem).wait()

        def addb(r, _):
            q = jnp.float32(0.25)
            lo = (stage[r, pl.ds(0, 16)] + zer[r, pl.ds(0, 16)]
                  + ga0[r, pl.ds(0, 16)] + ga1[r, pl.ds(0, 16)]) * q
            hi = (stage[r, pl.ds(16, 16)] + zer[r, pl.ds(16, 16)]
                  + ga0[r, pl.ds(16, 16)] + ga1[r, pl.ds(16, 16)]) * q
            stage[r, pl.ds(0, 16)] = lo
            stage[r, pl.ds(16, 16)] = hi
            return 0
        lax.fori_loop(0, WB, addb, 0)

        @pl.when(s < NS // 2)
        def _():
            pltpu.sync_copy(
                stage, outu.at[pl.ds(row0 + k * WB, WB), pl.ds(c * HALF, HALF)])

        @pl.when(s >= NS // 2)
        def _():
            pltpu.sync_copy(
                stage, outi.at[pl.ds(row0 - USER_N + k * WB, WB), pl.ds(c * HALF, HALF)])


@jax.jit
def _lgcn_sc(user_e, item_e, cols1, rows1, vals1):
    mesh = plsc.VectorSubcoreMesh(core_axis_name="c", subcore_axis_name="s")
    f = pl.kernel(
        _sc_body,
        out_type=[
            jax.ShapeDtypeStruct((USER_N, EMB), jnp.float32),
            jax.ShapeDtypeStruct((ITEM_N, EMB), jnp.float32),
            jax.ShapeDtypeStruct((2 * NODES, HALF), jnp.float32),
            jax.ShapeDtypeStruct((2 * NODES, HALF), jnp.float32),
            jax.ShapeDtypeStruct((2 * NODES, HALF), jnp.float32),
            jax.ShapeDtypeStruct((2 * NODES, HALF), jnp.float32),
        ],
        mesh=mesh,
        scratch_types=[
            pltpu.VMEM_SHARED((NODES, HALF), jnp.float32),
            pltpu.VMEM((BLK, HALF), jnp.float32),
            pltpu.VMEM((BLK, HALF), jnp.float32),
            pltpu.VMEM((BLK,), jnp.int32),
            pltpu.VMEM((BLK,), jnp.int32),
            pltpu.VMEM((BLK,), jnp.int32),
            pltpu.VMEM((BLK,), jnp.int32),
            pltpu.VMEM((CHUNK,), jnp.int32),
            pltpu.VMEM((BLK,), jnp.float32),
            pltpu.VMEM((BLK,), jnp.float32),
            pltpu.VMEM((WB, HALF), jnp.float32),
            pltpu.VMEM((WB, HALF), jnp.float32),
            pltpu.SemaphoreType.DMA,
            pltpu.SemaphoreType.DMA,
            pltpu.SemaphoreType.DMA,
            pltpu.SemaphoreType.DMA,
            pltpu.SemaphoreType.DMA,
            pltpu.SemaphoreType.DMA,
            pltpu.SemaphoreType.DMA,
        ],
        compiler_params=pltpu.CompilerParams(
            use_tc_tiling_on_sc=False, needs_layout_passes=False),
    )
    return f(user_e, item_e, cols1, rows1, vals1)


def kernel(user_emb, item_emb, user_prototypes, item_prototypes, adj_indices, adj_values):
    outs = _lgcn_sc(user_emb, item_emb, adj_indices[1], adj_indices[0], adj_values)
    return (outs[0], outs[1], user_emb, item_emb,
            user_prototypes, item_prototypes)


# speedup vs baseline: 14.6433x; 1.3204x over previous
import jax
import jax.numpy as jnp
from jax import lax
from jax.experimental import pallas as pl
from jax.experimental.pallas import tpu as pltpu
from jax.experimental.pallas import tpu_sc as plsc

USER_N = 25000
ITEM_N = 25000
NODES = USER_N + ITEM_N
EMB = 64
HALF = EMB // 2
LAYERS = 3
EDGES = 800000
NC = 2
NS = 16
CHUNK = 128
BLK = 2 * CHUNK
NCHT = EDGES // CHUNK
NB = 195
NTAIL = NCHT - NS * 2 * NB
STRIPE = NODES // NS
WB = 125
NWB = STRIPE // WB


def _zero2d(ref, nrows):
    def body(r, _):
        ref[r, pl.ds(0, 16)] = jnp.zeros((16,), jnp.float32)
        ref[r, pl.ds(16, 16)] = jnp.zeros((16,), jnp.float32)
        return 0
    lax.fori_loop(0, nrows, body, 0)


def _sc_body(user_e, item_e, cols1, rows1, vals1, outu, outi, x0b, x1, x2, x3,
             acc, ga0, ga1, cb0, cb1, rb0, rb1, rbt, vb0, vb1,
             zer, stage, gsem0, gsem1, ssem0, ssem1, msem, isem0, isem1,
             rsem0, rsem1):
    c = lax.axis_index("c")
    s = lax.axis_index("s")
    row0 = s * STRIPE
    cbase = s * (2 * NB) + jnp.minimum(s, NTAIL)
    coff = jnp.full((16,), c * NODES, jnp.int32)

    _zero2d(zer, WB)

    gas = (ga0, ga1)
    cbs = (cb0, cb1)
    rbs = (rb0, rb1)
    vbs = (vb0, vb1)
    gsems = (gsem0, gsem1)
    ssems = (ssem0, ssem1)
    isems = (isem0, isem1)
    rsems = (rsem0, rsem1)

    def drain_scat(p):
        pltpu.make_async_copy(gas[p], acc.at[rbs[p]], ssems[p]).wait()

    def fire(xi, j, p, first):
        if not first:
            drain_scat(p)
        blk = cbase + 2 * j
        off = blk * CHUNK
        if first:
            pltpu.sync_copy(cols1.at[pl.ds(off, BLK)], cbs[p])
            pltpu.sync_copy(vals1.at[pl.ds(off, BLK)], vbs[p])
        else:
            pltpu.make_async_copy(cols1.at[pl.ds(off, BLK)], cbs[p], isems[p]).wait()
            pltpu.make_async_copy(vals1.at[pl.ds(off, BLK)], vbs[p], isems[p]).wait()
        for g in range(BLK // 16):
            cbs[p][pl.ds(g * 16, 16)] = cbs[p][pl.ds(g * 16, 16)] + coff
        pltpu.async_copy(xi.at[cbs[p]], gas[p], gsems[p])
        pltpu.async_copy(rows1.at[pl.ds(off, BLK)], rbs[p], rsems[p])

    def scale(ga, vb, nedge):
        @plsc.parallel_loop(0, nedge // 16, step=1, unroll=2)
        def grp(g):
            vv = vb[pl.ds(g * 16, 16)]
            for e in range(16):
                j = g * 16 + e
                bcast = vv[jnp.full((16,), e, jnp.int32)]
                ga[j, pl.ds(0, 16)] = ga[j, pl.ds(0, 16)] * bcast
                ga[j, pl.ds(16, 16)] = ga[j, pl.ds(16, 16)] * bcast

    def process(xi, p, pf_j=None):
        pltpu.make_async_copy(xi.at[cbs[p]], gas[p], gsems[p]).wait()
        if pf_j is not None:
            pfoff = (cbase + 2 * pf_j) * CHUNK
            pltpu.async_copy(cols1.at[pl.ds(pfoff, BLK)], cbs[p], isems[p])
        scale(gas[p], vbs[p], BLK)
        if pf_j is not None:
            pfoff = (cbase + 2 * pf_j) * CHUNK
            pltpu.async_copy(vals1.at[pl.ds(pfoff, BLK)], vbs[p], isems[p])
        pltpu.make_async_copy(rows1.at[pl.ds(0, BLK)], rbs[p], rsems[p]).wait()
        pltpu.async_copy(gas[p], acc.at[rbs[p]], ssems[p], add=True)

    ga0v = ga0.at[pl.ds(0, WB)]
    bufs = (stage, ga0v)
    half = NS // 2
    for k in range(NWB):
        buf = bufs[k % 2]
        n = row0 + k * WB
        if k >= 2:
            pn = row0 + (k - 2) * WB
            pltpu.make_async_copy(
                bufs[k % 2], x0b.at[pl.ds(c * NODES + pn, WB)], ssem0).wait()

        @pl.when(s < half)
        def _():
            pltpu.sync_copy(user_e.at[pl.ds(n, WB), pl.ds(c * HALF, HALF)], buf)

        @pl.when(s >= half)
        def _():
            pltpu.sync_copy(item_e.at[pl.ds(n - USER_N, WB), pl.ds(c * HALF, HALF)], buf)
        pltpu.async_copy(buf, x0b.at[pl.ds(c * NODES + n, WB)], ssem0)
        pltpu.async_copy(zer, acc.at[pl.ds(n, WB)], gsem0)
    for k in range(NWB):
        pltpu.make_async_copy(zer, acc.at[pl.ds(row0 + k * WB, WB)], gsem0).wait()
    for k in (NWB - 2, NWB - 1):
        pltpu.make_async_copy(
            bufs[k % 2], x0b.at[pl.ds(c * NODES + row0 + k * WB, WB)], ssem0).wait()
    plsc.subcore_barrier()

    xs_in = (x0b, x1, x2)
    xs_out = (x1, x2, x3)
    for l in range(LAYERS):
        xi = xs_in[l]
        xo = xs_out[l]
        fire(xi, 0, 0, True)
        fire(xi, 1, 1, True)
        process(xi, 0, 2)

        def body(i2, _):
            j = 2 * i2
            fire(xi, j + 2, 0, False)
            process(xi, 1, j + 3)
            fire(xi, j + 3, 1, False)
            process(xi, 0, j + 4)
            return 0
        lax.fori_loop(0, (NB - 3) // 2, body, 0)
        fire(xi, NB - 1, 0, False)
        process(xi, 1)
        process(xi, 0)
        drain_scat(1)
        drain_scat(0)

        @pl.when(s < NTAIL)
        def _():
            blk = cbase + 2 * NB
            off = blk * CHUNK
            pltpu.sync_copy(cols1.at[pl.ds(off, CHUNK)], cb0.at[pl.ds(0, CHUNK)])
            for g in range(CHUNK // 16):
                cb0[pl.ds(g * 16, 16)] = cb0[pl.ds(g * 16, 16)] + coff
            pltpu.sync_copy(rows1.at[pl.ds(off, CHUNK)], rbt)
            pltpu.sync_copy(vals1.at[pl.ds(off, CHUNK)], vb0.at[pl.ds(0, CHUNK)])
            pltpu.async_copy(
                xi.at[cb0.at[pl.ds(0, CHUNK)]], ga0.at[pl.ds(0, CHUNK)], gsem0).wait()
            scale(ga0, vb0, CHUNK)
            pltpu.sync_copy(ga0.at[pl.ds(0, CHUNK)], acc.at[rbt], add=True)

        plsc.subcore_barrier()

        for k in range(NWB):
            buf = bufs[k % 2]
            b = row0 + k * WB
            if k >= 2:
                pltpu.make_async_copy(
                    buf, xo.at[pl.ds(c * NODES + b - 2 * WB, WB)], ssem0).wait()
            pltpu.sync_copy(acc.at[pl.ds(b, WB)], buf)
            if l < LAYERS - 1:
                pltpu.async_copy(zer, acc.at[pl.ds(b, WB)], gsem0)
            pltpu.async_copy(buf, xo.at[pl.ds(c * NODES + b, WB)], ssem0)
        for k in (NWB - 2, NWB - 1):
            pltpu.make_async_copy(
                bufs[k % 2], xo.at[pl.ds(c * NODES + row0 + k * WB, WB)], ssem0).wait()
        if l < LAYERS - 1:
            for k in range(NWB):
                pltpu.make_async_copy(
                    zer, acc.at[pl.ds(row0 + k * WB, WB)], gsem0).wait()
        plsc.subcore_barrier()

    ga1v = ga1.at[pl.ds(0, WB)]
    for k in range(NWB):
        b = c * NODES + row0 + k * WB
        pltpu.async_copy(x0b.at[pl.ds(b, WB)], stage, gsem0)
        pltpu.async_copy(x1.at[pl.ds(b, WB)], zer, gsem1)
        pltpu.async_copy(x2.at[pl.ds(b, WB)], ga0v, ssem1)
        pltpu.async_copy(x3.at[pl.ds(b, WB)], ga1v, msem)
        pltpu.make_async_copy(x0b.at[pl.ds(b, WB)], stage, gsem0).wait()
        pltpu.make_async_copy(x1.at[pl.ds(b, WB)], zer, gsem1).wait()
        pltpu.make_async_copy(x2.at[pl.ds(b, WB)], ga0v, ssem1).wait()
        pltpu.make_async_copy(x3.at[pl.ds(b, WB)], ga1v, msem).wait()

        def addb(r, _):
            q = jnp.float32(0.25)
            lo = (stage[r, pl.ds(0, 16)] + zer[r, pl.ds(0, 16)]
                  + ga0[r, pl.ds(0, 16)] + ga1[r, pl.ds(0, 16)]) * q
            hi = (stage[r, pl.ds(16, 16)] + zer[r, pl.ds(16, 16)]
                  + ga0[r, pl.ds(16, 16)] + ga1[r, pl.ds(16, 16)]) * q
            stage[r, pl.ds(0, 16)] = lo
            stage[r, pl.ds(16, 16)] = hi
            return 0
        lax.fori_loop(0, WB, addb, 0)

        @pl.when(s < NS // 2)
        def _():
            pltpu.sync_copy(
                stage, outu.at[pl.ds(row0 + k * WB, WB), pl.ds(c * HALF, HALF)])

        @pl.when(s >= NS // 2)
        def _():
            pltpu.sync_copy(
                stage, outi.at[pl.ds(row0 - USER_N + k * WB, WB), pl.ds(c * HALF, HALF)])


@jax.jit
def _lgcn_sc(user_e, item_e, cols1, rows1, vals1):
    mesh = plsc.VectorSubcoreMesh(core_axis_name="c", subcore_axis_name="s")
    f = pl.kernel(
        _sc_body,
        out_type=[
            jax.ShapeDtypeStruct((USER_N, EMB), jnp.float32),
            jax.ShapeDtypeStruct((ITEM_N, EMB), jnp.float32),
            jax.ShapeDtypeStruct((2 * NODES, HALF), jnp.float32),
            jax.ShapeDtypeStruct((2 * NODES, HALF), jnp.float32),
            jax.ShapeDtypeStruct((2 * NODES, HALF), jnp.float32),
            jax.ShapeDtypeStruct((2 * NODES, HALF), jnp.float32),
        ],
        mesh=mesh,
        scratch_types=[
            pltpu.VMEM_SHARED((NODES, HALF), jnp.float32),
            pltpu.VMEM((BLK, HALF), jnp.float32),
            pltpu.VMEM((BLK, HALF), jnp.float32),
            pltpu.VMEM((BLK,), jnp.int32),
            pltpu.VMEM((BLK,), jnp.int32),
            pltpu.VMEM((BLK,), jnp.int32),
            pltpu.VMEM((BLK,), jnp.int32),
            pltpu.VMEM((CHUNK,), jnp.int32),
            pltpu.VMEM((BLK,), jnp.float32),
            pltpu.VMEM((BLK,), jnp.float32),
            pltpu.VMEM((WB, HALF), jnp.float32),
            pltpu.VMEM((WB, HALF), jnp.float32),
            pltpu.SemaphoreType.DMA,
            pltpu.SemaphoreType.DMA,
            pltpu.SemaphoreType.DMA,
            pltpu.SemaphoreType.DMA,
            pltpu.SemaphoreType.DMA,
            pltpu.SemaphoreType.DMA,
            pltpu.SemaphoreType.DMA,
            pltpu.SemaphoreType.DMA,
            pltpu.SemaphoreType.DMA,
        ],
        compiler_params=pltpu.CompilerParams(
            use_tc_tiling_on_sc=False, needs_layout_passes=False),
    )
    return f(user_e, item_e, cols1, rows1, vals1)


def kernel(user_emb, item_emb, user_prototypes, item_prototypes, adj_indices, adj_values):
    outs = _lgcn_sc(user_emb, item_emb, adj_indices[1], adj_indices[0], adj_values)
    return (outs[0], outs[1], user_emb, item_emb,
            user_prototypes, item_prototypes)
